# Initial kernel scaffold; baseline (speedup 1.0000x reference)
import jax
import jax.numpy as jnp
from jax.experimental import pallas as pl


def kernel(inputs, edge_index, W0, al0, ar0, b0, W1, al1, ar1, b1, resW1):
    raise NotImplementedError("write your pallas kernel here")


# speedup vs baseline: 48.0409x; 48.0409x over previous
import functools

import jax
import jax.numpy as jnp
from jax import lax
from jax.experimental import pallas as pl
from jax.experimental.pallas import tpu as pltpu
from jax.experimental.pallas import tpu_sc as plsc

N = 10000
E = 320000
SLOPE = 0.2
NC = 2
NS = 16
NW = NC * NS
C = 128
NCHUNKS = E // C
RPT = 624
RPT_LAST = N - 15 * RPT

_f32 = jnp.float32
_i32 = jnp.int32


_B = 2000


def _pre_body(x_ref, w_ref, al_ref, ar_ref, t0_ref, er_ref):
    feat = jnp.dot(x_ref[...], w_ref[...], preferred_element_type=_f32)
    el = jnp.dot(feat, al_ref[...], preferred_element_type=_f32)
    er = jnp.dot(feat, ar_ref[...], preferred_element_type=_f32)
    t0_ref[...] = jnp.concatenate(
        [feat, el, jnp.zeros((_B, 8), _f32)], axis=1)
    er_ref[...] = er


def _tc_pre(x, W0, al0, ar0):
    grid = (N // _B,)
    return pl.pallas_call(
        _pre_body,
        grid=grid,
        in_specs=[
            pl.BlockSpec((_B, 128), lambda g: (g, 0)),
            pl.BlockSpec((128, 64), lambda g: (0, 0)),
            pl.BlockSpec((64, 8), lambda g: (0, 0)),
            pl.BlockSpec((64, 8), lambda g: (0, 0)),
        ],
        out_specs=[
            pl.BlockSpec((_B, 80), lambda g: (g, 0)),
            pl.BlockSpec((_B, 8), lambda g: (g, 0)),
        ],
        out_shape=[
            jax.ShapeDtypeStruct((N, 80), _f32),
            jax.ShapeDtypeStruct((N, 8), _f32),
        ],
    )(x, W0, al0, ar0)


def _mid_body(p_ref, rep_ref, b0_ref, w1_ref, al1_ref, ar1_ref, rw_ref,
              t1_ref, er_ref, res_ref):
    a = p_ref[0] + p_ref[1]
    esum = a[:, 64:72]
    denom = jnp.dot(esum, rep_ref[...], preferred_element_type=_f32) + 1e-9
    h = jax.nn.relu(a[:, :64] / denom + b0_ref[...])
    feat1 = jnp.dot(h, w1_ref[...], preferred_element_type=_f32)
    el1 = jnp.dot(feat1, al1_ref[...], preferred_element_type=_f32)
    er1 = jnp.dot(feat1, ar1_ref[...], preferred_element_type=_f32)
    res = jnp.dot(h, rw_ref[...], preferred_element_type=_f32)
    t1_ref[...] = jnp.concatenate(
        [feat1, el1, jnp.zeros((_B, 7), _f32)], axis=1)
    er_ref[...] = er1
    res_ref[...] = res


def _tc_mid(parts0, rep, b0, W1, al1, ar1, resW1):
    grid = (N // _B,)
    return pl.pallas_call(
        _mid_body,
        grid=grid,
        in_specs=[
            pl.BlockSpec((2, _B, 80), lambda g: (0, g, 0)),
            pl.BlockSpec((8, 64), lambda g: (0, 0)),
            pl.BlockSpec((1, 64), lambda g: (0, 0)),
            pl.BlockSpec((64, 40), lambda g: (0, 0)),
            pl.BlockSpec((40, 1), lambda g: (0, 0)),
            pl.BlockSpec((40, 1), lambda g: (0, 0)),
            pl.BlockSpec((64, 40), lambda g: (0, 0)),
        ],
        out_specs=[
            pl.BlockSpec((_B, 48), lambda g: (g, 0)),
            pl.BlockSpec((_B, 1), lambda g: (g, 0)),
            pl.BlockSpec((_B, 40), lambda g: (g, 0)),
        ],
        out_shape=[
            jax.ShapeDtypeStruct((N, 48), _f32),
            jax.ShapeDtypeStruct((N, 1), _f32),
            jax.ShapeDtypeStruct((N, 40), _f32),
        ],
    )(parts0, rep, b0, W1, al1, ar1, resW1)


def _post_body(q_ref, res_ref, b1_ref, out_ref):
    a = q_ref[0] + q_ref[1]
    out = a[:, :40] / (a[:, 40:41] + 1e-9)
    out_ref[...] = out + res_ref[...] + b1_ref[...]


def _tc_post(parts1, res, b1):
    grid = (N // _B,)
    return pl.pallas_call(
        _post_body,
        grid=grid,
        in_specs=[
            pl.BlockSpec((2, _B, 48), lambda g: (0, g, 0)),
            pl.BlockSpec((_B, 40), lambda g: (g, 0)),
            pl.BlockSpec((1, 40), lambda g: (0, 0)),
        ],
        out_specs=pl.BlockSpec((_B, 40), lambda g: (g, 0)),
        out_shape=jax.ShapeDtypeStruct((N, 40), _f32),
    )(parts1, res, b1)


_MESH = plsc.VectorSubcoreMesh(core_axis_name="c", subcore_axis_name="s")


def _init_acc(sid, z_hbm, acc):
    @pl.when(sid < 15)
    def _():
        pltpu.sync_copy(z_hbm.at[pl.ds(0, RPT)], acc.at[pl.ds(sid * RPT, RPT)])

    @pl.when(sid == 15)
    def _():
        pltpu.sync_copy(z_hbm, acc.at[pl.ds(15 * RPT, RPT_LAST)])


def _drain_acc(sid, cid, acc, out_hbm):
    @pl.when(sid < 15)
    def _():
        sl = pl.ds(sid * RPT, RPT)
        pltpu.sync_copy(acc.at[sl], out_hbm.at[cid].at[sl])

    @pl.when(sid == 15)
    def _():
        sl = pl.ds(15 * RPT, RPT_LAST)
        pltpu.sync_copy(acc.at[sl], out_hbm.at[cid].at[sl])


def _worker_chunks(wid):
    rem = NCHUNKS % NW
    return jnp.where(wid < rem, NCHUNKS // NW + 1, NCHUNKS // NW)


def _make_sc_edge0():
    @functools.partial(
        pl.kernel,
        out_type=jax.ShapeDtypeStruct((NC, N, 80), _f32),
        mesh=_MESH,
        compiler_params=pltpu.CompilerParams(needs_layout_passes=False, use_tc_tiling_on_sc=False),
        scratch_types=[
            pltpu.VMEM((1, C), _i32),
            pltpu.VMEM((1, C), _i32),
            pltpu.VMEM((C, 80), _f32),
            pltpu.VMEM((C * 8,), _f32),
            pltpu.VMEM((N // 2, 8), _i32),
            pltpu.VMEM_SHARED((N, 80), _f32),
            pltpu.SemaphoreType.DMA,
        ],
    )
    def sc_edge0(t0_hbm, src_hbm, dst_hbm, er_hbm, z_hbm, out_hbm,
                 srcv, dstv, rows, wv, erv, acc, sem):
        cid = lax.axis_index("c")
        sid = lax.axis_index("s")
        wid = cid * NS + sid
        lane = lax.iota(_i32, 16)
        hi8 = lane >> 3
        lo8 = lane & 7
        masklo = jnp.where(lane < 8, 1.0, 0.0).astype(_f32)
        zeros16 = jnp.zeros((16,), _i32)

        pltpu.sync_copy(er_hbm, erv)
        _init_acc(sid, z_hbm, acc)
        plsc.subcore_barrier()

        nch = _worker_chunks(wid)

        @pl.loop(0, nch)
        def _chunk(k):
            base = (wid + k * NW) * C
            pltpu.sync_copy(src_hbm.at[pl.ds(base, C)], srcv.at[0])
            pltpu.sync_copy(dst_hbm.at[pl.ds(base, C)], dstv.at[0])
            pltpu.async_copy(t0_hbm.at[srcv.at[0]], rows, sem).wait()

            @pl.loop(0, C // 2)
            def _w(g):
                rsel = 2 * g + hi8
                el = plsc.load_gather(rows, [rsel, 64 + lo8])
                dsel = plsc.load_gather(dstv, [zeros16, rsel])
                widx = dsel * 4 + (lo8 >> 1)
                word = plsc.load_gather(erv, [widx >> 3, widx & 7])
                shifted = jnp.where((lane & 1) == 1,
                                    word & jnp.int32(-65536), word << 16)
                er = plsc.bitcast(shifted, _f32)
                s = el + er
                w = jnp.exp(jnp.maximum(s, SLOPE * s))
                plsc.store_scatter(wv, [g * 16 + lane], w)

            @pl.loop(0, C)
            def _m(e):
                efull = jnp.full((16,), e, _i32)
                wbase = e * 8
                for j in range(4):
                    col = 16 * j + lane
                    v = plsc.load_gather(rows, [efull, col])
                    wp = plsc.load_gather(wv, [wbase + 2 * j + hi8])
                    plsc.store_scatter(rows, [efull, col], v * wp)
                wt = plsc.load_gather(wv, [wbase + lo8]) * masklo
                plsc.store_scatter(rows, [efull, 64 + lane], wt)

            pltpu.sync_copy(rows, acc.at[dstv.at[0]], add=True)

        plsc.subcore_barrier()
        _drain_acc(sid, cid, acc, out_hbm)

    return sc_edge0


def _make_sc_edge1():
    @functools.partial(
        pl.kernel,
        out_type=jax.ShapeDtypeStruct((NC, N, 48), _f32),
        mesh=_MESH,
        compiler_params=pltpu.CompilerParams(needs_layout_passes=False, use_tc_tiling_on_sc=False),
        scratch_types=[
            pltpu.VMEM((1, C), _i32),
            pltpu.VMEM((1, C), _i32),
            pltpu.VMEM((C, 48), _f32),
            pltpu.VMEM((C,), _f32),
            pltpu.VMEM((N // 8, 8), _f32),
            pltpu.VMEM_SHARED((N, 48), _f32),
            pltpu.SemaphoreType.DMA,
        ],
    )
    def sc_edge1(t1_hbm, src_hbm, dst_hbm, er_hbm, z_hbm, out_hbm,
                 srcv, dstv, rows, wv, erv, acc, sem):
        cid = lax.axis_index("c")
        sid = lax.axis_index("s")
        wid = cid * NS + sid
        lane = lax.iota(_i32, 16)
        zeros16 = jnp.zeros((16,), _i32)
        c40 = jnp.full((16,), 40, _i32)

        pltpu.sync_copy(er_hbm, erv)
        _init_acc(sid, z_hbm, acc)
        plsc.subcore_barrier()

        nch = _worker_chunks(wid)

        @pl.loop(0, nch)
        def _chunk(k):
            base = (wid + k * NW) * C
            pltpu.sync_copy(src_hbm.at[pl.ds(base, C)], srcv.at[0])
            pltpu.sync_copy(dst_hbm.at[pl.ds(base, C)], dstv.at[0])
            pltpu.async_copy(t1_hbm.at[srcv.at[0]], rows, sem).wait()

            @pl.loop(0, C // 16)
            def _w(g):
                rsel = g * 16 + lane
                el = plsc.load_gather(rows, [rsel, c40])
                dsel = plsc.load_gather(dstv, [zeros16, rsel])
                er = plsc.load_gather(erv, [dsel >> 3, dsel & 7])
                s = el + er
                w = jnp.exp(jnp.maximum(s, SLOPE * s))
                plsc.store_scatter(wv, [rsel], w)

            @pl.loop(0, C // 16)
            def _m(g):
                for kk in range(16):
                    e = g * 16 + kk
                    efull = jnp.full((16,), e, _i32)
                    wp = plsc.load_gather(wv, [efull])
                    for j in range(3):
                        col = 16 * j + lane
                        v = plsc.load_gather(rows, [efull, col])
                        plsc.store_scatter(rows, [efull, col], v * wp)
                rsel = g * 16 + lane
                w16 = plsc.load_gather(wv, [rsel])
                plsc.store_scatter(rows, [rsel, c40], w16)

            pltpu.sync_copy(rows, acc.at[dstv.at[0]], add=True)

        plsc.subcore_barrier()
        _drain_acc(sid, cid, acc, out_hbm)

    return sc_edge1


_SC_EDGE0 = _make_sc_edge0()
_SC_EDGE1 = _make_sc_edge1()


def kernel(inputs, edge_index, W0, al0, ar0, b0, W1, al1, ar1, b1, resW1):
    src = edge_index[0]
    dst = edge_index[1]
    eye8 = jnp.eye(8, dtype=_f32)
    al0m = (al0[:, :, None] * eye8[:, None, :]).reshape(64, 8)
    ar0m = (ar0[:, :, None] * eye8[:, None, :]).reshape(64, 8)
    rep = jnp.repeat(eye8, 8, axis=1).reshape(8, 64)
    t0, er0 = _tc_pre(inputs, W0, al0m, ar0m)
    er0p = jax.lax.bitcast_convert_type(
        er0.astype(jnp.bfloat16).reshape(N, 4, 2), _i32).reshape(N // 2, 8)
    z0 = jnp.zeros((RPT_LAST, 80), _f32)
    parts0 = _SC_EDGE0(t0, src, dst, er0p, z0)
    t1, er1p, res = _tc_mid(parts0, rep, b0[None], W1,
                            al1.reshape(40, 1), ar1.reshape(40, 1), resW1)
    er1p = er1p.reshape(N // 8, 8)
    z1 = jnp.zeros((RPT_LAST, 48), _f32)
    parts1 = _SC_EDGE1(t1, src, dst, er1p, z1)
    return _tc_post(parts1, res, b1[None])


# speedup vs baseline: 56.2737x; 1.1714x over previous
import functools

import jax
import jax.numpy as jnp
from jax import lax
from jax.experimental import pallas as pl
from jax.experimental.pallas import tpu as pltpu
from jax.experimental.pallas import tpu_sc as plsc

N = 10000
E = 320000
SLOPE = 0.2
NC = 2
NS = 16
NW = NC * NS
C = 128
NCHUNKS = E // C
RPT = 624
RPT_LAST = N - 15 * RPT

_f32 = jnp.float32
_i32 = jnp.int32


_B = 2000


def _pre_body(x_ref, w_ref, al_ref, ar_ref, t0_ref, er_ref):
    feat = jnp.dot(x_ref[...], w_ref[...], preferred_element_type=_f32)
    el = jnp.dot(feat, al_ref[...], preferred_element_type=_f32)
    er = jnp.dot(feat, ar_ref[...], preferred_element_type=_f32)
    t0_ref[...] = jnp.concatenate(
        [feat, el, jnp.zeros((_B, 8), _f32)], axis=1)
    er_ref[...] = er


def _tc_pre(x, W0, al0, ar0):
    grid = (N // _B,)
    return pl.pallas_call(
        _pre_body,
        grid=grid,
        in_specs=[
            pl.BlockSpec((_B, 128), lambda g: (g, 0)),
            pl.BlockSpec((128, 64), lambda g: (0, 0)),
            pl.BlockSpec((64, 8), lambda g: (0, 0)),
            pl.BlockSpec((64, 8), lambda g: (0, 0)),
        ],
        out_specs=[
            pl.BlockSpec((_B, 80), lambda g: (g, 0)),
            pl.BlockSpec((_B, 8), lambda g: (g, 0)),
        ],
        out_shape=[
            jax.ShapeDtypeStruct((N, 80), _f32),
            jax.ShapeDtypeStruct((N, 8), _f32),
        ],
    )(x, W0, al0, ar0)


def _mid_body(p_ref, rep_ref, b0_ref, w1_ref, al1_ref, ar1_ref, rw_ref,
              t1_ref, er_ref, res_ref):
    a = p_ref[0] + p_ref[1]
    esum = a[:, 64:72]
    denom = jnp.dot(esum, rep_ref[...], preferred_element_type=_f32) + 1e-9
    h = jax.nn.relu(a[:, :64] / denom + b0_ref[...])
    feat1 = jnp.dot(h, w1_ref[...], preferred_element_type=_f32)
    el1 = jnp.dot(feat1, al1_ref[...], preferred_element_type=_f32)
    er1 = jnp.dot(feat1, ar1_ref[...], preferred_element_type=_f32)
    res = jnp.dot(h, rw_ref[...], preferred_element_type=_f32)
    t1_ref[...] = jnp.concatenate(
        [feat1, el1, jnp.zeros((_B, 7), _f32)], axis=1)
    er_ref[...] = er1
    res_ref[...] = res


def _tc_mid(parts0, rep, b0, W1, al1, ar1, resW1):
    grid = (N // _B,)
    return pl.pallas_call(
        _mid_body,
        grid=grid,
        in_specs=[
            pl.BlockSpec((2, _B, 80), lambda g: (0, g, 0)),
            pl.BlockSpec((8, 64), lambda g: (0, 0)),
            pl.BlockSpec((1, 64), lambda g: (0, 0)),
            pl.BlockSpec((64, 40), lambda g: (0, 0)),
            pl.BlockSpec((40, 1), lambda g: (0, 0)),
            pl.BlockSpec((40, 1), lambda g: (0, 0)),
            pl.BlockSpec((64, 40), lambda g: (0, 0)),
        ],
        out_specs=[
            pl.BlockSpec((_B, 48), lambda g: (g, 0)),
            pl.BlockSpec((_B, 1), lambda g: (g, 0)),
            pl.BlockSpec((_B, 40), lambda g: (g, 0)),
        ],
        out_shape=[
            jax.ShapeDtypeStruct((N, 48), _f32),
            jax.ShapeDtypeStruct((N, 1), _f32),
            jax.ShapeDtypeStruct((N, 40), _f32),
        ],
    )(parts0, rep, b0, W1, al1, ar1, resW1)


def _post_body(q_ref, res_ref, b1_ref, out_ref):
    a = q_ref[0] + q_ref[1]
    out = a[:, :40] / (a[:, 40:41] + 1e-9)
    out_ref[...] = out + res_ref[...] + b1_ref[...]


def _tc_post(parts1, res, b1):
    grid = (N // _B,)
    return pl.pallas_call(
        _post_body,
        grid=grid,
        in_specs=[
            pl.BlockSpec((2, _B, 48), lambda g: (0, g, 0)),
            pl.BlockSpec((_B, 40), lambda g: (g, 0)),
            pl.BlockSpec((1, 40), lambda g: (0, 0)),
        ],
        out_specs=pl.BlockSpec((_B, 40), lambda g: (g, 0)),
        out_shape=jax.ShapeDtypeStruct((N, 40), _f32),
    )(parts1, res, b1)


_MESH = plsc.VectorSubcoreMesh(core_axis_name="c", subcore_axis_name="s")


def _init_acc(sid, z_hbm, acc):
    @pl.when(sid < 15)
    def _():
        pltpu.sync_copy(z_hbm.at[pl.ds(0, RPT)], acc.at[pl.ds(sid * RPT, RPT)])

    @pl.when(sid == 15)
    def _():
        pltpu.sync_copy(z_hbm, acc.at[pl.ds(15 * RPT, RPT_LAST)])


def _drain_acc(sid, cid, acc, out_hbm):
    @pl.when(sid < 15)
    def _():
        sl = pl.ds(sid * RPT, RPT)
        pltpu.sync_copy(acc.at[sl], out_hbm.at[cid].at[sl])

    @pl.when(sid == 15)
    def _():
        sl = pl.ds(15 * RPT, RPT_LAST)
        pltpu.sync_copy(acc.at[sl], out_hbm.at[cid].at[sl])


def _worker_chunks(wid):
    rem = NCHUNKS % NW
    return jnp.where(wid < rem, NCHUNKS // NW + 1, NCHUNKS // NW)


def _make_sc_edge0():
    @functools.partial(
        pl.kernel,
        out_type=jax.ShapeDtypeStruct((NC, N, 80), _f32),
        mesh=_MESH,
        compiler_params=pltpu.CompilerParams(needs_layout_passes=False, use_tc_tiling_on_sc=False),
        scratch_types=[
            pltpu.VMEM((2, C), _i32),
            pltpu.VMEM((2, C), _i32),
            pltpu.VMEM((2, C, 80), _f32),
            pltpu.VMEM((C * 8,), _f32),
            pltpu.VMEM((N // 2, 8), _i32),
            pltpu.VMEM_SHARED((N, 80), _f32),
            pltpu.SemaphoreType.DMA,
            pltpu.SemaphoreType.DMA,
            pltpu.SemaphoreType.DMA,
            pltpu.SemaphoreType.DMA,
        ],
    )
    def sc_edge0(t0_hbm, src_hbm, dst_hbm, er_hbm, z_hbm, out_hbm,
                 srcv, dstv, rows, wv, erv, acc,
                 gsem0, gsem1, ssem0, ssem1):
        cid = lax.axis_index("c")
        sid = lax.axis_index("s")
        wid = cid * NS + sid
        lane = lax.iota(_i32, 16)
        hi8 = lane >> 3
        lo8 = lane & 7
        masklo = jnp.where(lane < 8, 1.0, 0.0).astype(_f32)

        pltpu.sync_copy(er_hbm, erv)
        _init_acc(sid, z_hbm, acc)
        plsc.subcore_barrier()

        nch = _worker_chunks(wid)
        gsem = (gsem0, gsem1)
        ssem = (ssem0, ssem1)

        def issue(k2, p):
            base = (wid + k2 * NW) * C
            pltpu.sync_copy(src_hbm.at[pl.ds(base, C)], srcv.at[p])
            pltpu.sync_copy(dst_hbm.at[pl.ds(base, C)], dstv.at[p])
            pltpu.async_copy(t0_hbm.at[srcv.at[p]], rows.at[p], gsem[p])

        def wait_scatter(p):
            pltpu.make_async_copy(
                rows.at[p], acc.at[dstv.at[p]], ssem[p]).wait()

        def body(k, p):
            np_ = 1 - p
            rp = rows.at[p]
            dp = dstv.at[p]

            @pl.when(k + 1 < nch)
            def _():
                @pl.when(k >= 1)
                def _():
                    wait_scatter(np_)
                issue(k + 1, np_)

            pltpu.make_async_copy(t0_hbm.at[srcv.at[p]], rp, gsem[p]).wait()

            @pl.loop(0, C // 2)
            def _w(g):
                rsel = 2 * g + hi8
                el = plsc.load_gather(rp, [rsel, 64 + lo8])
                dsel = plsc.load_gather(dp, [rsel])
                widx = dsel * 4 + (lo8 >> 1)
                word = plsc.load_gather(erv, [widx >> 3, widx & 7])
                shifted = jnp.where((lane & 1) == 1,
                                    word & jnp.int32(-65536), word << 16)
                er = plsc.bitcast(shifted, _f32)
                s = el + er
                w = jnp.exp(jnp.maximum(s, SLOPE * s))
                plsc.store_scatter(wv, [g * 16 + lane], w)

            @pl.loop(0, C)
            def _m(e):
                efull = jnp.full((16,), e, _i32)
                wbase = e * 8
                for j in range(4):
                    col = 16 * j + lane
                    v = plsc.load_gather(rp, [efull, col])
                    wp = plsc.load_gather(wv, [wbase + 2 * j + hi8])
                    plsc.store_scatter(rp, [efull, col], v * wp)
                wt = plsc.load_gather(wv, [wbase + lo8]) * masklo
                plsc.store_scatter(rp, [efull, 64 + lane], wt)

            pltpu.async_copy(rp, acc.at[dp], ssem[p], add=True)

        issue(0, 0)

        @pl.loop(0, nch)
        def _chunk(k):
            @pl.when(k % 2 == 0)
            def _():
                body(k, 0)

            @pl.when(k % 2 == 1)
            def _():
                body(k, 1)

        wait_scatter(0)
        wait_scatter(1)
        plsc.subcore_barrier()
        _drain_acc(sid, cid, acc, out_hbm)

    return sc_edge0


def _make_sc_edge1():
    @functools.partial(
        pl.kernel,
        out_type=jax.ShapeDtypeStruct((NC, N, 48), _f32),
        mesh=_MESH,
        compiler_params=pltpu.CompilerParams(needs_layout_passes=False, use_tc_tiling_on_sc=False),
        scratch_types=[
            pltpu.VMEM((2, C), _i32),
            pltpu.VMEM((2, C), _i32),
            pltpu.VMEM((2, C, 48), _f32),
            pltpu.VMEM((C,), _f32),
            pltpu.VMEM((N // 8, 8), _f32),
            pltpu.VMEM_SHARED((N, 48), _f32),
            pltpu.SemaphoreType.DMA,
            pltpu.SemaphoreType.DMA,
            pltpu.SemaphoreType.DMA,
            pltpu.SemaphoreType.DMA,
        ],
    )
    def sc_edge1(t1_hbm, src_hbm, dst_hbm, er_hbm, z_hbm, out_hbm,
                 srcv, dstv, rows, wv, erv, acc,
                 gsem0, gsem1, ssem0, ssem1):
        cid = lax.axis_index("c")
        sid = lax.axis_index("s")
        wid = cid * NS + sid
        lane = lax.iota(_i32, 16)
        c40 = jnp.full((16,), 40, _i32)

        pltpu.sync_copy(er_hbm, erv)
        _init_acc(sid, z_hbm, acc)
        plsc.subcore_barrier()

        nch = _worker_chunks(wid)
        gsem = (gsem0, gsem1)
        ssem = (ssem0, ssem1)

        def issue(k2, p):
            base = (wid + k2 * NW) * C
            pltpu.sync_copy(src_hbm.at[pl.ds(base, C)], srcv.at[p])
            pltpu.sync_copy(dst_hbm.at[pl.ds(base, C)], dstv.at[p])
            pltpu.async_copy(t1_hbm.at[srcv.at[p]], rows.at[p], gsem[p])

        def wait_scatter(p):
            pltpu.make_async_copy(
                rows.at[p], acc.at[dstv.at[p]], ssem[p]).wait()

        def body(k, p):
            np_ = 1 - p
            rp = rows.at[p]
            dp = dstv.at[p]

            @pl.when(k + 1 < nch)
            def _():
                @pl.when(k >= 1)
                def _():
                    wait_scatter(np_)
                issue(k + 1, np_)

            pltpu.make_async_copy(t1_hbm.at[srcv.at[p]], rp, gsem[p]).wait()

            @pl.loop(0, C // 16)
            def _w(g):
                rsel = g * 16 + lane
                el = plsc.load_gather(rp, [rsel, c40])
                dsel = plsc.load_gather(dp, [rsel])
                er = plsc.load_gather(erv, [dsel >> 3, dsel & 7])
                s = el + er
                w = jnp.exp(jnp.maximum(s, SLOPE * s))
                plsc.store_scatter(wv, [rsel], w)

            @pl.loop(0, C // 16)
            def _m(g):
                for kk in range(16):
                    e = g * 16 + kk
                    efull = jnp.full((16,), e, _i32)
                    wp = plsc.load_gather(wv, [efull])
                    for j in range(3):
                        col = 16 * j + lane
                        v = plsc.load_gather(rp, [efull, col])
                        plsc.store_scatter(rp, [efull, col], v * wp)
                rsel = g * 16 + lane
                w16 = plsc.load_gather(wv, [rsel])
                plsc.store_scatter(rp, [rsel, c40], w16)

            pltpu.async_copy(rp, acc.at[dp], ssem[p], add=True)

        issue(0, 0)

        @pl.loop(0, nch)
        def _chunk(k):
            @pl.when(k % 2 == 0)
            def _():
                body(k, 0)

            @pl.when(k % 2 == 1)
            def _():
                body(k, 1)

        wait_scatter(0)
        wait_scatter(1)
        plsc.subcore_barrier()
        _drain_acc(sid, cid, acc, out_hbm)

    return sc_edge1


_SC_EDGE0 = _make_sc_edge0()
_SC_EDGE1 = _make_sc_edge1()


def kernel(inputs, edge_index, W0, al0, ar0, b0, W1, al1, ar1, b1, resW1):
    src = edge_index[0]
    dst = edge_index[1]
    eye8 = jnp.eye(8, dtype=_f32)
    al0m = (al0[:, :, None] * eye8[:, None, :]).reshape(64, 8)
    ar0m = (ar0[:, :, None] * eye8[:, None, :]).reshape(64, 8)
    rep = jnp.repeat(eye8, 8, axis=1).reshape(8, 64)
    t0, er0 = _tc_pre(inputs, W0, al0m, ar0m)
    er0p = jax.lax.bitcast_convert_type(
        er0.astype(jnp.bfloat16).reshape(N, 4, 2), _i32).reshape(N // 2, 8)
    z0 = jnp.zeros((RPT_LAST, 80), _f32)
    parts0 = _SC_EDGE0(t0, src, dst, er0p, z0)
    t1, er1p, res = _tc_mid(parts0, rep, b0[None], W1,
                            al1.reshape(40, 1), ar1.reshape(40, 1), resW1)
    er1p = er1p.reshape(N // 8, 8)
    z1 = jnp.zeros((RPT_LAST, 48), _f32)
    parts1 = _SC_EDGE1(t1, src, dst, er1p, z1)
    return _tc_post(parts1, res, b1[None])
